# baseline (speedup 1.0000x reference)
import functools

import jax
import jax.numpy as jnp
from jax import lax
from jax.experimental import pallas as pl
from jax.experimental.pallas import tpu as pltpu
from jax.experimental.pallas import tpu_sc as plsc

P = 262144
K = 1000000
D = 32
NC = 2
NS = 16
NW = NC * NS
PW = P // NW
C = 16
G = C * 8
NCHUNK = PW // C
NBUF = 4
NLOOP = NCHUNK // NBUF

TB = 128
NFULL = K // TB
TAIL = K - NFULL * TB
NTB = NFULL + 1
TROUND = (NTB + NW - 1) // NW

_mesh = plsc.VectorSubcoreMesh(core_axis_name="c", subcore_axis_name="s")


@functools.partial(
    pl.kernel,
    mesh=_mesh,
    out_type=jax.ShapeDtypeStruct((K, D), jnp.float32),
    compiler_params=pltpu.CompilerParams(
        use_tc_tiling_on_sc=False, needs_layout_passes=False),
    scratch_types=[
        pltpu.VMEM((2, D, 133), jnp.float32),
        pltpu.VMEM((2, TB, D), jnp.float32),
        pltpu.SemaphoreType.DMA,
        pltpu.SemaphoreType.DMA,
        pltpu.SemaphoreType.DMA,
        pltpu.SemaphoreType.DMA,
    ],
)
def _table_transpose(tt_hbm, out_hbm, stage_v, tout_v, s0, s1, o0, o1):
    ssem = (s0, s1)
    osem = (o0, o1)
    wid = lax.axis_index("s") * NC + lax.axis_index("c")
    iota = lax.iota(jnp.int32, 16)

    def stage_copy(blk, b, w):
        return pltpu.make_async_copy(
            tt_hbm.at[:, pl.ds(blk * TB, w)], stage_v.at[b, :, pl.ds(0, w)],
            ssem[b])

    def out_copy(blk, b, w):
        return pltpu.make_async_copy(
            tout_v.at[b, pl.ds(0, w), :], out_hbm.at[pl.ds(blk * TB, w), :],
            osem[b])

    def transpose_block(b, w):
        for j in range(w):
            lo = plsc.load_gather(
                stage_v.at[b], [iota, jnp.full((16,), j, jnp.int32)])
            hi = plsc.load_gather(
                stage_v.at[b], [iota + 16, jnp.full((16,), j, jnp.int32)])
            tout_v[b, j, pl.ds(0, 16)] = lo
            tout_v[b, j, pl.ds(16, 16)] = hi

    def blk_of(t):
        return wid + NW * t

    @pl.when(blk_of(0) < NTB)
    def _():
        stage_copy(blk_of(0), 0, TB).start()

    @pl.when(blk_of(1) < NTB)
    def _():
        stage_copy(blk_of(1), 1, TB).start()

    def start_next(t, bb):
        nxt = blk_of(t + 2)

        @pl.when(nxt < NFULL)
        def _():
            stage_copy(nxt, bb, TB).start()

        @pl.when(nxt == NFULL)
        def _():
            stage_copy(nxt, bb, TAIL).start()

    def loop_body(t, carry):
        blk = blk_of(t)
        b = lax.rem(t, 2)
        for bb in range(2):
            @pl.when((b == bb) & (blk < NFULL))
            def _():
                stage_copy(0, bb, TB).wait()

                @pl.when(t >= 2)
                def _():
                    out_copy(0, bb, TB).wait()

                transpose_block(bb, TB)
                out_copy(blk, bb, TB).start()
                start_next(t, bb)

            @pl.when((b == bb) & (blk == NFULL))
            def _():
                stage_copy(0, bb, TAIL).wait()

                @pl.when(t >= 2)
                def _():
                    out_copy(0, bb, TB).wait()

                transpose_block(bb, TAIL)
                out_copy(blk, bb, TAIL).start()

        return carry

    lax.fori_loop(0, TROUND, loop_body, 0)
    out_copy(0, 1, TB).wait()
    last_even = blk_of(TROUND - 1)

    @pl.when(last_even == NFULL)
    def _():
        out_copy(0, 0, TAIL).wait()

    @pl.when(last_even != NFULL)
    def _():
        out_copy(0, 0, TB).wait()


@functools.partial(
    pl.kernel,
    mesh=_mesh,
    out_type=jax.ShapeDtypeStruct((P, D), jnp.float32),
    compiler_params=pltpu.CompilerParams(
        use_tc_tiling_on_sc=False, needs_layout_passes=False),
    scratch_types=[
        pltpu.VMEM((NBUF, G), jnp.int32),
        pltpu.VMEM((NBUF, 3, C), jnp.float32),
        pltpu.VMEM((NBUF, G, D), jnp.float32),
        pltpu.VMEM((NBUF, C, D), jnp.float32),
        pltpu.SemaphoreType.DMA,
        pltpu.SemaphoreType.DMA,
        pltpu.SemaphoreType.DMA,
        pltpu.SemaphoreType.DMA,
        pltpu.SemaphoreType.DMA,
        pltpu.SemaphoreType.DMA,
        pltpu.SemaphoreType.DMA,
        pltpu.SemaphoreType.DMA,
        pltpu.SemaphoreType.DMA,
        pltpu.SemaphoreType.DMA,
        pltpu.SemaphoreType.DMA,
        pltpu.SemaphoreType.DMA,
        pltpu.SemaphoreType.DMA,
        pltpu.SemaphoreType.DMA,
        pltpu.SemaphoreType.DMA,
        pltpu.SemaphoreType.DMA,
    ],
)
def _voxel_interp(featsT_hbm, pt_hbm, table_hbm, out_hbm,
                  idx_v, p_v, rows_v, out_v,
                  i0, i1, i2, i3, p0, p1, p2, p3,
                  g0, g1, g2, g3, o0, o1, o2, o3):
    isem = (i0, i1, i2, i3)
    psem = (p0, p1, p2, p3)
    gsem = (g0, g1, g2, g3)
    osem = (o0, o1, o2, o3)
    wid = lax.axis_index("s") * NC + lax.axis_index("c")
    base0 = wid * PW

    def idx_copy(g, b, c):
        return pltpu.make_async_copy(
            featsT_hbm.at[c, pl.ds(base0 + g * C, C)],
            idx_v.at[b, pl.ds(c * C, C)], isem[b])

    def p_copy(g, b, j):
        return pltpu.make_async_copy(
            pt_hbm.at[j, pl.ds(base0 + g * C, C)], p_v.at[b, j], psem[b])

    def gather_copy(b):
        return pltpu.make_async_copy(
            table_hbm.at[idx_v.at[b]], rows_v.at[b], gsem[b])

    def out_copy(g, b):
        return pltpu.make_async_copy(
            out_v.at[b], out_hbm.at[pl.ds(base0 + g * C, C)], osem[b])

    def stage_in(g, b):
        for c in range(8):
            idx_copy(g, b, c).start()
        for j in range(3):
            p_copy(g, b, j).start()

    def wait_idx(b):
        for c in range(8):
            idx_copy(0, b, c).wait()

    def compute(b):
        px = p_v[b, 0, pl.ds(0, 16)]
        py = p_v[b, 1, pl.ds(0, 16)]
        pz = p_v[b, 2, pl.ds(0, 16)]
        wx = (1.0 - px, px)
        wy = (1.0 - py, py)
        wz = (1.0 - pz, pz)
        wxy = (wx[0] * wy[0], wx[0] * wy[1], wx[1] * wy[0], wx[1] * wy[1])
        wvec = tuple(wxy[c >> 1] * wz[c & 1] for c in range(8))
        for i in range(C):
            acc0 = None
            acc1 = None
            for c in range(8):
                w = wvec[c][i]
                t0 = w * rows_v[b, c * C + i, pl.ds(0, 16)]
                t1 = w * rows_v[b, c * C + i, pl.ds(16, 16)]
                acc0 = t0 if acc0 is None else acc0 + t0
                acc1 = t1 if acc1 is None else acc1 + t1
            out_v[b, i, pl.ds(0, 16)] = acc0
            out_v[b, i, pl.ds(16, 16)] = acc1

    for b in range(NBUF):
        stage_in(b, b)
    for b in range(NBUF):
        wait_idx(b)
        gather_copy(b).start()

    def loop_body(it, carry):
        for b in range(NBUF):
            g = NBUF * it + b
            gather_copy(b).wait()

            @pl.when(g + NBUF < NCHUNK)
            def _():
                for c in range(8):
                    idx_copy(g + NBUF, b, c).start()

            for j in range(3):
                p_copy(g, b, j).wait()

            @pl.when(it > 0)
            def _():
                out_copy(g, b).wait()

            compute(b)
            out_copy(g, b).start()

            @pl.when(g + NBUF < NCHUNK)
            def _():
                for j in range(3):
                    p_copy(g + NBUF, b, j).start()
                wait_idx(b)
                gather_copy(b).start()

        return carry

    lax.fori_loop(0, NLOOP, loop_body, 0)
    for b in range(NBUF):
        out_copy(0, b).wait()


def kernel(feats, p, values_weight):
    table = _table_transpose(values_weight.T)
    return _voxel_interp(feats.T, p.T, table)


# speedup vs baseline: 3.4247x; 3.4247x over previous
import functools

import jax
import jax.numpy as jnp
from jax import lax
from jax.experimental import pallas as pl
from jax.experimental.pallas import tpu as pltpu
from jax.experimental.pallas import tpu_sc as plsc

P = 262144
K = 1000000
D = 32
NC = 2
NS = 16
NW = NC * NS
PW = P // NW
C = 16
CP = C + 1
G = C * 8
NCHUNK = PW // C
NBUF = 6
NLOOP = NCHUNK // NBUF
NTAIL = NCHUNK - NLOOP * NBUF

_mesh = plsc.VectorSubcoreMesh(core_axis_name="c", subcore_axis_name="s")


@functools.partial(
    pl.kernel,
    mesh=_mesh,
    out_type=jax.ShapeDtypeStruct((D, P), jnp.float32),
    compiler_params=pltpu.CompilerParams(
        use_tc_tiling_on_sc=False, needs_layout_passes=False),
    scratch_types=[
        pltpu.VMEM((NBUF, G), jnp.int32),
        pltpu.VMEM((NBUF, 3, C), jnp.float32),
        pltpu.VMEM((NBUF, G, D), jnp.float32),
        pltpu.VMEM((NBUF, D, CP), jnp.float32),
        [pltpu.SemaphoreType.DMA] * NBUF,
        [pltpu.SemaphoreType.DMA] * NBUF,
        [pltpu.SemaphoreType.DMA] * NBUF,
        [pltpu.SemaphoreType.DMA] * NBUF,
    ],
)
def _voxel_interp(featsT_hbm, pt_hbm, table_hbm, out_hbm,
                  idx_v, p_v, rows_v, out_v,
                  isem, psem, gsem, osem):
    wid = lax.axis_index("s") * NC + lax.axis_index("c")
    base0 = wid * PW
    iota = lax.iota(jnp.int32, 16)

    def idx_copy(g, b, c):
        return pltpu.make_async_copy(
            featsT_hbm.at[c, pl.ds(base0 + g * C, C)],
            idx_v.at[b, pl.ds(c * C, C)], isem[b])

    def p_copy(g, b, j):
        return pltpu.make_async_copy(
            pt_hbm.at[j, pl.ds(base0 + g * C, C)], p_v.at[b, j], psem[b])

    def gather_copy(b):
        return pltpu.make_async_copy(
            table_hbm.at[idx_v.at[b]], rows_v.at[b], gsem[b])

    def out_copy(g, b):
        return pltpu.make_async_copy(
            out_v.at[b, :, pl.ds(0, C)],
            out_hbm.at[:, pl.ds(base0 + g * C, C)], osem[b])

    def stage_in(g, b):
        for c in range(8):
            idx_copy(g, b, c).start()
        for j in range(3):
            p_copy(g, b, j).start()

    def wait_idx(b):
        for c in range(8):
            idx_copy(0, b, c).wait()

    def compute(b):
        px = p_v[b, 0, pl.ds(0, 16)]
        py = p_v[b, 1, pl.ds(0, 16)]
        pz = p_v[b, 2, pl.ds(0, 16)]
        wx = (1.0 - px, px)
        wy = (1.0 - py, py)
        wz = (1.0 - pz, pz)
        wxy = (wx[0] * wy[0], wx[0] * wy[1], wx[1] * wy[0], wx[1] * wy[1])
        wvec = tuple(wxy[c >> 1] * wz[c & 1] for c in range(8))
        for i in range(C):
            acc0 = None
            acc1 = None
            for c in range(8):
                w = wvec[c][i]
                t0 = w * rows_v[b, c * C + i, pl.ds(0, 16)]
                t1 = w * rows_v[b, c * C + i, pl.ds(16, 16)]
                acc0 = t0 if acc0 is None else acc0 + t0
                acc1 = t1 if acc1 is None else acc1 + t1
            coli = jnp.full((16,), i, jnp.int32)
            plsc.store_scatter(out_v.at[b], [iota, coli], acc0)
            plsc.store_scatter(out_v.at[b], [iota + 16, coli], acc1)

    for b in range(NBUF):
        stage_in(b, b)
    for b in range(NBUF):
        wait_idx(b)
        gather_copy(b).start()

    def loop_body(it, carry):
        for b in range(NBUF):
            g = NBUF * it + b
            gather_copy(b).wait()

            @pl.when(g + NBUF < NCHUNK)
            def _():
                for c in range(8):
                    idx_copy(g + NBUF, b, c).start()

            for j in range(3):
                p_copy(g, b, j).wait()

            @pl.when(it > 0)
            def _():
                out_copy(g, b).wait()

            compute(b)
            out_copy(g, b).start()

            @pl.when(g + NBUF < NCHUNK)
            def _():
                for j in range(3):
                    p_copy(g + NBUF, b, j).start()
                wait_idx(b)
                gather_copy(b).start()

        return carry

    lax.fori_loop(0, NLOOP, loop_body, 0)
    for b in range(NTAIL):
        g = NLOOP * NBUF + b
        gather_copy(b).wait()
        for j in range(3):
            p_copy(0, b, j).wait()
        out_copy(0, b).wait()
        compute(b)
        out_copy(g, b).start()
    for b in range(NTAIL, NBUF):
        out_copy(0, b).wait()
    for b in range(NTAIL):
        out_copy(0, b).wait()


def kernel(feats, p, values_weight):
    return _voxel_interp(feats.T, p.T, values_weight).T


# speedup vs baseline: 3.5709x; 1.0427x over previous
import functools

import jax
import jax.numpy as jnp
from jax import lax
from jax.experimental import pallas as pl
from jax.experimental.pallas import tpu as pltpu
from jax.experimental.pallas import tpu_sc as plsc

P = 262144
K = 1000000
D = 32
NC = 2
NS = 16
NW = NC * NS
PW = P // NW
C = 32
G = C * 8
NG = G // 128
NCHUNK = PW // C
NBUF = 4
NLOOP = NCHUNK // NBUF

_mesh = plsc.VectorSubcoreMesh(core_axis_name="c", subcore_axis_name="s")


@functools.partial(
    pl.kernel,
    mesh=_mesh,
    out_type=jax.ShapeDtypeStruct((P, D), jnp.float32),
    compiler_params=pltpu.CompilerParams(
        use_tc_tiling_on_sc=False, needs_layout_passes=False),
    scratch_types=[
        pltpu.VMEM((NBUF, NG, 128), jnp.int32),
        pltpu.VMEM((NBUF, 3, C), jnp.float32),
        pltpu.VMEM((NBUF, G, D), jnp.float32),
        pltpu.VMEM((NBUF, C, D), jnp.float32),
        [pltpu.SemaphoreType.DMA] * NBUF,
        [pltpu.SemaphoreType.DMA] * NBUF,
        [pltpu.SemaphoreType.DMA] * NBUF,
        [pltpu.SemaphoreType.DMA] * NBUF,
    ],
)
def _voxel_interp(featsT_hbm, pt_hbm, table_hbm, out_hbm,
                  idx_v, p_v, rows_v, out_v,
                  isem, psem, gsem, osem):
    wid = lax.axis_index("s") * NC + lax.axis_index("c")
    base0 = wid * PW

    def idx_copy(g, b, c):
        return pltpu.make_async_copy(
            featsT_hbm.at[c, pl.ds(base0 + g * C, C)],
            idx_v.at[b, (c * C) // 128, pl.ds((c * C) % 128, C)], isem[b])

    def p_copy(g, b, j):
        return pltpu.make_async_copy(
            pt_hbm.at[j, pl.ds(base0 + g * C, C)], p_v.at[b, j], psem[b])

    def gather_copy(b, k):
        return pltpu.make_async_copy(
            table_hbm.at[idx_v.at[b, k]],
            rows_v.at[b, pl.ds(k * 128, 128)], gsem[b])

    def out_copy(g, b):
        return pltpu.make_async_copy(
            out_v.at[b], out_hbm.at[pl.ds(base0 + g * C, C)], osem[b])

    def stage_in(g, b):
        for c in range(8):
            idx_copy(g, b, c).start()
        for j in range(3):
            p_copy(g, b, j).start()

    def wait_idx(b):
        for c in range(8):
            idx_copy(0, b, c).wait()

    def start_gathers(b):
        for k in range(NG):
            gather_copy(b, k).start()

    def wait_gathers(b):
        for k in range(NG):
            gather_copy(b, k).wait()

    def compute(b):
        for h in range(2):
            px = p_v[b, 0, pl.ds(16 * h, 16)]
            py = p_v[b, 1, pl.ds(16 * h, 16)]
            pz = p_v[b, 2, pl.ds(16 * h, 16)]
            wx = (1.0 - px, px)
            wy = (1.0 - py, py)
            wz = (1.0 - pz, pz)
            wxy = (wx[0] * wy[0], wx[0] * wy[1], wx[1] * wy[0], wx[1] * wy[1])
            wvec = tuple(wxy[c >> 1] * wz[c & 1] for c in range(8))
            for ii in range(16):
                i = 16 * h + ii
                acc0 = None
                acc1 = None
                for c in range(8):
                    w = wvec[c][ii]
                    t0 = w * rows_v[b, c * C + i, pl.ds(0, 16)]
                    t1 = w * rows_v[b, c * C + i, pl.ds(16, 16)]
                    acc0 = t0 if acc0 is None else acc0 + t0
                    acc1 = t1 if acc1 is None else acc1 + t1
                out_v[b, i, pl.ds(0, 16)] = acc0
                out_v[b, i, pl.ds(16, 16)] = acc1

    for b in range(NBUF):
        stage_in(b, b)
    for b in range(NBUF):
        wait_idx(b)
        start_gathers(b)

    def loop_body(it, carry):
        for b in range(NBUF):
            g = NBUF * it + b
            wait_gathers(b)

            @pl.when(g + NBUF < NCHUNK)
            def _():
                for c in range(8):
                    idx_copy(g + NBUF, b, c).start()

            for j in range(3):
                p_copy(g, b, j).wait()

            @pl.when(it > 0)
            def _():
                out_copy(g, b).wait()

            compute(b)
            out_copy(g, b).start()

            @pl.when(g + NBUF < NCHUNK)
            def _():
                for j in range(3):
                    p_copy(g + NBUF, b, j).start()
                wait_idx(b)
                start_gathers(b)

        return carry

    lax.fori_loop(0, NLOOP, loop_body, 0)
    for b in range(NBUF):
        out_copy(0, b).wait()


def kernel(feats, p, values_weight):
    return _voxel_interp(feats.T, p.T, values_weight)
